# baseline (speedup 1.0000x reference)
import jax
import jax.numpy as jnp
from jax import lax
from jax.experimental import pallas as pl
from jax.experimental.pallas import tpu as pltpu
from jax.experimental.pallas import tpu_sc as plsc

N = 10000
D = 128
E = 320000
E_SELF = E + N
NC, NS, L = 2, 16, 16
NW = NC * NS
C = 32
CH = 324
EPW = C * CH
E_PAD = EPW * NW
E_ARR = E_PAD
N_ACC = 10112
RPT = N_ACC // NS
_ROW_SPLITS = (64, 64, 64, 64, 64, 64, 64, 64, 64, 56)


def _edge_body(xl_hbm, xr_hbm, att_hbm, src_hbm, dst_hbm, out_hbm, den_hbm,
               src0, src1, src2, src3, dst0, dst1, dst2, dst3,
               xl0, xl1, xl2, xl3, xr0, xr1, xr2, xr3, attbuf, denom_v,
               acc, sl0, sl1, sl2, sl3, sr0, sr1, sr2, sr3,
               ss0, ss1, ss2, ss3):
    cid = lax.axis_index("c")
    sid = lax.axis_index("s")
    wid = sid * NC + cid
    srcs, dsts = (src0, src1, src2, src3), (dst0, dst1, dst2, dst3)
    xls, xrs = (xl0, xl1, xl2, xl3), (xr0, xr1, xr2, xr3)
    sls, srs = (sl0, sl1, sl2, sl3), (sr0, sr1, sr2, sr3)
    sss = (ss0, ss1, ss2, ss3)

    zeros16 = jnp.zeros((L,), jnp.float32)

    def zbody(i, _):
        for j in range(D // L):
            xl0[i, pl.ds(j * L, L)] = zeros16
        return 0

    lax.fori_loop(0, C, zbody, 0)


    def zdbody(i, _):
        denom_v[pl.ds(i * L, L)] = zeros16
        return 0

    lax.fori_loop(0, N_ACC // L, zdbody, 0)

    r0 = sid * RPT
    off = 0
    for nrow in _ROW_SPLITS:
        pltpu.sync_copy(xl0.at[pl.ds(0, nrow), :],
                        acc.at[pl.ds(r0 + off, nrow), :])
        off += nrow

    pltpu.sync_copy(att_hbm, attbuf)
    plsc.subcore_barrier()

    att_js = [attbuf[pl.ds(j * L, L)] for j in range(D // L)]
    lane0 = lax.iota(jnp.int32, L) == 0
    ebase = wid * EPW

    def issue(g, b):
        base = ebase + g * C
        pltpu.sync_copy(src_hbm.at[pl.ds(base, C)], srcs[b])
        pltpu.sync_copy(dst_hbm.at[pl.ds(base, C)], dsts[b])
        pltpu.async_copy(xl_hbm.at[srcs[b]], xls[b], sls[b])
        pltpu.async_copy(xr_hbm.at[dsts[b]], xrs[b], srs[b])

    def compute(g, b):
        pltpu.make_async_copy(xl_hbm.at[srcs[b]], xls[b], sls[b]).wait()
        pltpu.make_async_copy(xr_hbm.at[dsts[b]], xrs[b], srs[b]).wait()
        xlbuf, xrbuf, dst_v = xls[b], xrs[b], dsts[b]
        base = ebase + g * C

        @plsc.parallel_loop(0, C, step=1, unroll=4)
        def edge_body(e):
            avs = []
            s = jnp.zeros((L,), jnp.float32)
            for j in range(D // L):
                a = xlbuf[e, pl.ds(j * L, L)]
                b_ = xrbuf[e, pl.ds(j * L, L)]
                avs.append(a)
                t = a + b_
                t = jnp.maximum(t, 0.2 * t)
                s = s + t * att_js[j]
            alpha = jnp.sum(s)
            gate = jnp.where(base + e < E_SELF, 1.0, 0.0)
            wv = jnp.exp(jnp.broadcast_to(alpha, (L,)))
            wv = wv * jnp.broadcast_to(gate, (L,))
            for j in range(D // L):
                xlbuf[e, pl.ds(j * L, L)] = avs[j] * wv
            dvec = plsc.load_gather(dst_v, [jnp.broadcast_to(e, (L,))])
            plsc.addupdate_scatter(denom_v, [dvec], wv, mask=lane0)

        pltpu.sync_copy(xlbuf, acc.at[dst_v], add=True)

    def swait(b):
        pass

    issue(0, 0)
    issue(1, 1)
    compute(0, 0)
    issue(2, 2)
    compute(1, 1)
    issue(3, 3)

    def quad_body(q, _):
        g = 2 + 4 * q
        for k in range(4):
            gk = g + k
            swait(k)
            issue(gk + 2, k)
            compute(gk, (2 + k) % 4)
        return 0

    lax.fori_loop(0, (CH - 4) // 4, quad_body, 0)

    compute(CH - 2, (CH - 2) % 4)
    compute(CH - 1, (CH - 1) % 4)
    for b in range(4):
        swait(b)

    pltpu.sync_copy(denom_v, den_hbm.at[cid, sid, :])
    plsc.subcore_barrier()
    off = 0
    for nrow in _ROW_SPLITS:
        pltpu.sync_copy(acc.at[pl.ds(r0 + off, nrow), :],
                        out_hbm.at[cid, pl.ds(r0 + off, nrow), :])
        off += nrow


_edge_call = pl.kernel(
    _edge_body,
    out_type=(jax.ShapeDtypeStruct((NC, N_ACC, D), jnp.float32),
              jax.ShapeDtypeStruct((NC, NS, N_ACC), jnp.float32)),
    mesh=plsc.VectorSubcoreMesh(core_axis_name="c", subcore_axis_name="s",
                                num_cores=NC, num_subcores=NS),
    compiler_params=pltpu.CompilerParams(needs_layout_passes=False),
    scratch_types=(
        [pltpu.VMEM((C,), jnp.int32)] * 8
        + [pltpu.VMEM((C, D), jnp.float32)] * 8
        + [pltpu.VMEM((D,), jnp.float32),
           pltpu.VMEM((N_ACC,), jnp.float32),
           pltpu.VMEM_SHARED((N_ACC, D), jnp.float32)]
        + [pltpu.SemaphoreType.DMA] * 12
    ),
)


BM = 1000


def _mm_body(x_ref, wl_ref, wr_ref, wres_ref, b_ref, xl_ref, xr_ref, res_ref):
    xb = x_ref[...]
    xl_ref[...] = jnp.dot(xb, wl_ref[...], preferred_element_type=jnp.float32)
    xr_ref[...] = jnp.dot(xb, wr_ref[...], preferred_element_type=jnp.float32)
    res_ref[...] = (jnp.dot(xb, wres_ref[...], preferred_element_type=jnp.float32)
                    + b_ref[...])


_mm_call = pl.pallas_call(
    _mm_body,
    grid=(N // BM,),
    in_specs=[
        pl.BlockSpec((BM, D), lambda i: (i, 0)),
        pl.BlockSpec((D, D), lambda i: (0, 0)),
        pl.BlockSpec((D, D), lambda i: (0, 0)),
        pl.BlockSpec((D, D), lambda i: (0, 0)),
        pl.BlockSpec((1, D), lambda i: (0, 0)),
    ],
    out_specs=[
        pl.BlockSpec((BM, D), lambda i: (i, 0)),
        pl.BlockSpec((BM, D), lambda i: (i, 0)),
        pl.BlockSpec((BM, D), lambda i: (i, 0)),
    ],
    out_shape=[jax.ShapeDtypeStruct((N, D), jnp.float32)] * 3,
)


def _fin_body(acc_ref, den_ref, res_ref, o_ref):
    num = acc_ref[0] + acc_ref[1]
    den = jnp.sum(den_ref[...], axis=(0, 1))[:, None] + 1e-16
    o_ref[...] = jnp.maximum(num / den + res_ref[...], 0.0)


BF = 128

_fin_call = pl.pallas_call(
    _fin_body,
    grid=(N_ACC // BF,),
    in_specs=[
        pl.BlockSpec((NC, BF, D), lambda i: (0, i, 0)),
        pl.BlockSpec((NC, NS, BF), lambda i: (0, 0, i)),
        pl.BlockSpec((BF, D), lambda i: (i, 0)),
    ],
    out_specs=pl.BlockSpec((BF, D), lambda i: (i, 0)),
    out_shape=jax.ShapeDtypeStruct((N, D), jnp.float32),
)


def kernel(x, edge_index, W_l, W_r, att, W_res, bias):
    x = x.astype(jnp.float32)
    src = edge_index[0].astype(jnp.int32)
    dst = edge_index[1].astype(jnp.int32)
    loop = jnp.arange(N, dtype=jnp.int32)
    pad = jnp.zeros((E_ARR - E_SELF,), jnp.int32)
    srcp = jnp.concatenate([src, loop, pad])
    dstp = jnp.concatenate([dst, loop, pad])
    xl, xr, res = _mm_call(x, W_l.T, W_r.T, W_res.T, bias.reshape(1, D))
    accs, dens = _edge_call(xl, xr, att.astype(jnp.float32), srcp, dstp)
    return _fin_call(accs, dens, res)


# speedup vs baseline: 1.2267x; 1.2267x over previous
import jax
import jax.numpy as jnp
from jax import lax
from jax.experimental import pallas as pl
from jax.experimental.pallas import tpu as pltpu
from jax.experimental.pallas import tpu_sc as plsc

N = 10000
D = 128
E = 320000
E_SELF = E + N
NC, NS, L = 2, 16, 16
NW = NC * NS
C = 96
CH = 108
EPW = C * CH
E_PAD = EPW * NW
E_ARR = E_PAD + C
N_ACC = 10112
RPT = N_ACC // NS
_ROW_SPLITS = (64, 64, 64, 64, 64, 64, 64, 64, 64, 56)


def _edge_body(xl_hbm, xr_hbm, att_hbm, idx_hbm, out_hbm, den_hbm,
               idx0, idx1, xl0, xl1, xr0, xr1, attbuf, denom_v,
               acc, sl0, sl1, sr0, sr1):
    cid = lax.axis_index("c")
    sid = lax.axis_index("s")
    wid = sid * NC + cid
    idxs = (idx0, idx1)
    xls, xrs = (xl0, xl1), (xr0, xr1)
    sls, srs = (sl0, sl1), (sr0, sr1)

    zeros16 = jnp.zeros((L,), jnp.float32)

    def zbody(i, _):
        for j in range(D // L):
            xl0[i, pl.ds(j * L, L)] = zeros16
        return 0

    lax.fori_loop(0, C, zbody, 0)

    def zdbody(i, _):
        denom_v[pl.ds(i * L, L)] = zeros16
        return 0

    lax.fori_loop(0, N_ACC // L, zdbody, 0)

    r0 = sid * RPT
    off = 0
    for nrow in _ROW_SPLITS:
        pltpu.sync_copy(xl0.at[pl.ds(0, nrow), :],
                        acc.at[pl.ds(r0 + off, nrow), :])
        off += nrow

    pltpu.sync_copy(att_hbm, attbuf)
    plsc.subcore_barrier()

    att_js = [attbuf[pl.ds(j * L, L)] for j in range(D // L)]
    lane0 = lax.iota(jnp.int32, L) == 0
    cbase = wid * CH

    def issue(g, b):
        pltpu.sync_copy(idx_hbm.at[cbase + g], idxs[b])
        pltpu.async_copy(xl_hbm.at[idxs[b].at[0]], xls[b], sls[b])
        pltpu.async_copy(xr_hbm.at[idxs[b].at[1]], xrs[b], srs[b])

    def compute(g, b):
        pltpu.make_async_copy(xl_hbm.at[idxs[b].at[0]], xls[b], sls[b]).wait()
        pltpu.make_async_copy(xr_hbm.at[idxs[b].at[1]], xrs[b], srs[b]).wait()
        xlbuf, xrbuf, idx_v = xls[b], xrs[b], idxs[b]
        base = (cbase + g) * C

        @plsc.parallel_loop(0, C, step=1, unroll=4)
        def edge_body(e):
            avs = []
            s = jnp.zeros((L,), jnp.float32)
            for j in range(D // (2 * L)):
                pw = xrbuf[e, pl.ds(j * L, L)]
                packed = plsc.bitcast(pw, jnp.bfloat16)
                b0, b1 = plsc.unpack(packed, format=plsc.PackFormat.INTERLEAVED)
                for h, bh in ((0, b0), (1, b1)):
                    jj = 2 * j + h
                    a = xlbuf[e, pl.ds(jj * L, L)]
                    avs.append(a)
                    t = a + bh
                    t = jnp.maximum(t, 0.2 * t)
                    s = s + t * att_js[jj]
            alpha = jnp.sum(s)
            gate = jnp.where(base + e < E_SELF, 1.0, 0.0)
            wv = jnp.exp(jnp.broadcast_to(alpha, (L,)))
            wv = wv * jnp.broadcast_to(gate, (L,))
            for j in range(D // L):
                xlbuf[e, pl.ds(j * L, L)] = avs[j] * wv
            dvec = plsc.load_gather(idx_v, [jnp.ones((L,), jnp.int32),
                                            jnp.broadcast_to(e, (L,))])
            plsc.addupdate_scatter(denom_v, [dvec], wv, mask=lane0)

        pltpu.sync_copy(xlbuf, acc.at[idx_v.at[1]], add=True)

    issue(0, 0)

    def pair_body(g2, _):
        g = 2 * g2
        issue(g + 1, 1)
        compute(g, 0)
        issue(g + 2, 0)
        compute(g + 1, 1)
        return 0

    lax.fori_loop(0, CH // 2, pair_body, 0)

    pltpu.make_async_copy(xl_hbm.at[idx0.at[0]], xl0, sl0).wait()
    pltpu.make_async_copy(xr_hbm.at[idx0.at[1]], xr0, sr0).wait()

    pltpu.sync_copy(denom_v, den_hbm.at[cid, sid, :])
    plsc.subcore_barrier()
    off = 0
    for nrow in _ROW_SPLITS:
        pltpu.sync_copy(acc.at[pl.ds(r0 + off, nrow), :],
                        out_hbm.at[cid, pl.ds(r0 + off, nrow), :])
        off += nrow


_edge_call = pl.kernel(
    _edge_body,
    out_type=(jax.ShapeDtypeStruct((NC, N_ACC, D), jnp.float32),
              jax.ShapeDtypeStruct((NC, NS, N_ACC), jnp.float32)),
    mesh=plsc.VectorSubcoreMesh(core_axis_name="c", subcore_axis_name="s",
                                num_cores=NC, num_subcores=NS),
    compiler_params=pltpu.CompilerParams(needs_layout_passes=False,
                                        use_tc_tiling_on_sc=False),
    scratch_types=[
        pltpu.VMEM((2, C), jnp.int32),
        pltpu.VMEM((2, C), jnp.int32),
        pltpu.VMEM((C, D), jnp.float32),
        pltpu.VMEM((C, D), jnp.float32),
        pltpu.VMEM((C, D // 2), jnp.int32),
        pltpu.VMEM((C, D // 2), jnp.int32),
        pltpu.VMEM((D,), jnp.float32),
        pltpu.VMEM((N_ACC,), jnp.float32),
        pltpu.VMEM_SHARED((N_ACC, D), jnp.float32),
        pltpu.SemaphoreType.DMA,
        pltpu.SemaphoreType.DMA,
        pltpu.SemaphoreType.DMA,
        pltpu.SemaphoreType.DMA,
    ],
)


BM = 1000


def _mm_body(x_ref, wl_ref, wr_ref, wres_ref, b_ref, xl_ref, xr_ref, res_ref):
    xb = x_ref[...]
    xl_ref[...] = jnp.dot(xb, wl_ref[...], preferred_element_type=jnp.float32)
    xr_ref[...] = jnp.dot(xb, wr_ref[...], preferred_element_type=jnp.float32)
    res_ref[...] = (jnp.dot(xb, wres_ref[...], preferred_element_type=jnp.float32)
                    + b_ref[...])


_mm_call = pl.pallas_call(
    _mm_body,
    grid=(N // BM,),
    in_specs=[
        pl.BlockSpec((BM, D), lambda i: (i, 0)),
        pl.BlockSpec((D, D), lambda i: (0, 0)),
        pl.BlockSpec((D, D), lambda i: (0, 0)),
        pl.BlockSpec((D, D), lambda i: (0, 0)),
        pl.BlockSpec((1, D), lambda i: (0, 0)),
    ],
    out_specs=[
        pl.BlockSpec((BM, D), lambda i: (i, 0)),
        pl.BlockSpec((BM, D), lambda i: (i, 0)),
        pl.BlockSpec((BM, D), lambda i: (i, 0)),
    ],
    out_shape=[jax.ShapeDtypeStruct((N, D), jnp.float32)] * 3,
)


def _fin_body(acc_ref, den_ref, res_ref, o_ref):
    num = acc_ref[0] + acc_ref[1]
    den = jnp.sum(den_ref[...], axis=(0, 1))[:, None] + 1e-16
    o_ref[...] = jnp.maximum(num / den + res_ref[...], 0.0)


BF = 128

_fin_call = pl.pallas_call(
    _fin_body,
    grid=(N_ACC // BF,),
    in_specs=[
        pl.BlockSpec((NC, BF, D), lambda i: (0, i, 0)),
        pl.BlockSpec((NC, NS, BF), lambda i: (0, 0, i)),
        pl.BlockSpec((BF, D), lambda i: (i, 0)),
    ],
    out_specs=pl.BlockSpec((BF, D), lambda i: (i, 0)),
    out_shape=jax.ShapeDtypeStruct((N, D), jnp.float32),
)


def kernel(x, edge_index, W_l, W_r, att, W_res, bias):
    x = x.astype(jnp.float32)
    src = edge_index[0].astype(jnp.int32)
    dst = edge_index[1].astype(jnp.int32)
    loop = jnp.arange(N, dtype=jnp.int32)
    pad = jnp.zeros((E_ARR - E_SELF,), jnp.int32)
    srcp = jnp.concatenate([src, loop, pad]).reshape(E_ARR // C, 1, C)
    dstp = jnp.concatenate([dst, loop, pad]).reshape(E_ARR // C, 1, C)
    idxp = jnp.concatenate([srcp, dstp], axis=1)
    xl, xr, res = _mm_call(x, W_l.T, W_r.T, W_res.T, bias.reshape(1, D))
    xr_sw = (xr.reshape(N, D // 32, 2, 16).swapaxes(2, 3)
             .reshape(N, D // 2, 2).astype(jnp.bfloat16))
    xr_sw = lax.bitcast_convert_type(xr_sw, jnp.int32)
    accs, dens = _edge_call(xl, xr_sw, att.astype(jnp.float32), idxp)
    return _fin_call(accs, dens, res)


# speedup vs baseline: 1.4819x; 1.2080x over previous
import jax
import jax.numpy as jnp
from jax import lax
from jax.experimental import pallas as pl
from jax.experimental.pallas import tpu as pltpu
from jax.experimental.pallas import tpu_sc as plsc

N = 10000
D = 128
E = 320000
E_SELF = E + N
NC, NS, L = 2, 16, 16
NW = NC * NS
C = 64
CH = 162
EPW = C * CH
E_PAD = EPW * NW
E_ARR = E_PAD + C
N_ACC = 10112
RPT = N_ACC // NS
_ROW_SPLITS = (64, 64, 64, 64, 64, 64, 64, 64, 64, 56)


def _edge_body(xl_hbm, xr_hbm, att_hbm, src_hbm, dst_hbm, out_hbm, den_hbm,
               src0, src1, dst0, dst1, xl0, xl1, xr0, xr1, attbuf, denom_v,
               wbuf, acc, sl0, sl1, sr0, sr1):
    cid = lax.axis_index("c")
    sid = lax.axis_index("s")
    wid = sid * NC + cid
    srcs, dsts = (src0, src1), (dst0, dst1)
    xls, xrs = (xl0, xl1), (xr0, xr1)
    sls, srs = (sl0, sl1), (sr0, sr1)

    zeros16 = jnp.zeros((L,), jnp.float32)

    def zbody(i, _):
        for j in range(D // L):
            xl0[i, pl.ds(j * L, L)] = zeros16
        return 0

    lax.fori_loop(0, C, zbody, 0)

    def zdbody(i, _):
        denom_v[pl.ds(i * L, L)] = zeros16
        return 0

    lax.fori_loop(0, N_ACC // L, zdbody, 0)

    r0 = sid * RPT
    off = 0
    for nrow in _ROW_SPLITS:
        pltpu.sync_copy(xl0.at[pl.ds(0, nrow), :],
                        acc.at[pl.ds(r0 + off, nrow), :])
        off += nrow

    pltpu.sync_copy(att_hbm, attbuf)
    plsc.subcore_barrier()

    att_js = [attbuf[pl.ds(j * L, L)] for j in range(D // L)]
    lane0 = lax.iota(jnp.int32, L) == 0
    ebase = wid * EPW

    def issue(g, b):
        base = ebase + g * C
        pltpu.sync_copy(src_hbm.at[pl.ds(base, C)], srcs[b])
        pltpu.sync_copy(dst_hbm.at[pl.ds(base, C)], dsts[b])
        pltpu.async_copy(xl_hbm.at[srcs[b]], xls[b], sls[b])
        pltpu.async_copy(xr_hbm.at[dsts[b]], xrs[b], srs[b])

    def compute(g, b):
        pltpu.make_async_copy(xl_hbm.at[srcs[b]], xls[b], sls[b]).wait()
        pltpu.make_async_copy(xr_hbm.at[dsts[b]], xrs[b], srs[b]).wait()
        xlbuf, xrbuf, dst_v = xls[b], xrs[b], dsts[b]
        base = ebase + g * C

        @plsc.parallel_loop(0, C, step=1, unroll=4)
        def edge_body(e):
            avs = []
            s = jnp.zeros((L,), jnp.float32)
            for j in range(D // L):
                a = xlbuf[e, pl.ds(j * L, L)]
                b_ = xrbuf[e, pl.ds(j * L, L)]
                avs.append(a)
                t = a + b_
                t = jnp.maximum(t, 0.2 * t)
                s = s + t * att_js[j]
            alpha = jnp.sum(s)
            gate = jnp.where(base + e < E_SELF, 1.0, 0.0)
            wv = jnp.exp(jnp.broadcast_to(alpha, (L,)))
            wv = wv * jnp.broadcast_to(gate, (L,))
            for j in range(D // L):
                xlbuf[e, pl.ds(j * L, L)] = avs[j] * wv
            plsc.store_scatter(wbuf, [jnp.broadcast_to(e, (L,))], wv,
                               mask=lane0)

        def den_body(k, _):
            w16 = wbuf[pl.ds(k * L, L)]
            d16 = dst_v[pl.ds(k * L, L)]
            plsc.addupdate_scatter(denom_v, [d16], w16)
            return 0

        lax.fori_loop(0, C // L, den_body, 0, unroll=3)

        pltpu.sync_copy(xlbuf, acc.at[dst_v], add=True)

    issue(0, 0)

    def pair_body(g2, _):
        g = 2 * g2
        issue(g + 1, 1)
        compute(g, 0)
        issue(g + 2, 0)
        compute(g + 1, 1)
        return 0

    lax.fori_loop(0, CH // 2, pair_body, 0)

    pltpu.make_async_copy(xl_hbm.at[src0], xl0, sl0).wait()
    pltpu.make_async_copy(xr_hbm.at[dst0], xr0, sr0).wait()

    pltpu.sync_copy(denom_v, den_hbm.at[cid, sid, :])
    plsc.subcore_barrier()
    off = 0
    for nrow in _ROW_SPLITS:
        pltpu.sync_copy(acc.at[pl.ds(r0 + off, nrow), :],
                        out_hbm.at[cid, pl.ds(r0 + off, nrow), :])
        off += nrow


_edge_call = pl.kernel(
    _edge_body,
    out_type=(jax.ShapeDtypeStruct((NC, N_ACC, D), jnp.float32),
              jax.ShapeDtypeStruct((NC, NS, N_ACC), jnp.float32)),
    mesh=plsc.VectorSubcoreMesh(core_axis_name="c", subcore_axis_name="s",
                                num_cores=NC, num_subcores=NS),
    compiler_params=pltpu.CompilerParams(needs_layout_passes=False),
    scratch_types=[
        pltpu.VMEM((C,), jnp.int32),
        pltpu.VMEM((C,), jnp.int32),
        pltpu.VMEM((C,), jnp.int32),
        pltpu.VMEM((C,), jnp.int32),
        pltpu.VMEM((C, D), jnp.float32),
        pltpu.VMEM((C, D), jnp.float32),
        pltpu.VMEM((C, D), jnp.float32),
        pltpu.VMEM((C, D), jnp.float32),
        pltpu.VMEM((D,), jnp.float32),
        pltpu.VMEM((N_ACC,), jnp.float32),
        pltpu.VMEM((C,), jnp.float32),
        pltpu.VMEM_SHARED((N_ACC, D), jnp.float32),
        pltpu.SemaphoreType.DMA,
        pltpu.SemaphoreType.DMA,
        pltpu.SemaphoreType.DMA,
        pltpu.SemaphoreType.DMA,
    ],
)


BM = 1000


def _mm_body(x_ref, wl_ref, wr_ref, wres_ref, b_ref, xl_ref, xr_ref, res_ref):
    xb = x_ref[...]
    xl_ref[...] = jnp.dot(xb, wl_ref[...], preferred_element_type=jnp.float32)
    xr_ref[...] = jnp.dot(xb, wr_ref[...], preferred_element_type=jnp.float32)
    res_ref[...] = (jnp.dot(xb, wres_ref[...], preferred_element_type=jnp.float32)
                    + b_ref[...])


_mm_call = pl.pallas_call(
    _mm_body,
    grid=(N // BM,),
    in_specs=[
        pl.BlockSpec((BM, D), lambda i: (i, 0)),
        pl.BlockSpec((D, D), lambda i: (0, 0)),
        pl.BlockSpec((D, D), lambda i: (0, 0)),
        pl.BlockSpec((D, D), lambda i: (0, 0)),
        pl.BlockSpec((1, D), lambda i: (0, 0)),
    ],
    out_specs=[
        pl.BlockSpec((BM, D), lambda i: (i, 0)),
        pl.BlockSpec((BM, D), lambda i: (i, 0)),
        pl.BlockSpec((BM, D), lambda i: (i, 0)),
    ],
    out_shape=[jax.ShapeDtypeStruct((N, D), jnp.float32)] * 3,
)


def _fin_body(acc_ref, den_ref, res_ref, o_ref):
    num = acc_ref[0] + acc_ref[1]
    den = jnp.sum(den_ref[...], axis=(0, 1))[:, None] + 1e-16
    o_ref[...] = jnp.maximum(num / den + res_ref[...], 0.0)


BF = 128

_fin_call = pl.pallas_call(
    _fin_body,
    grid=(N_ACC // BF,),
    in_specs=[
        pl.BlockSpec((NC, BF, D), lambda i: (0, i, 0)),
        pl.BlockSpec((NC, NS, BF), lambda i: (0, 0, i)),
        pl.BlockSpec((BF, D), lambda i: (i, 0)),
    ],
    out_specs=pl.BlockSpec((BF, D), lambda i: (i, 0)),
    out_shape=jax.ShapeDtypeStruct((N, D), jnp.float32),
)


def kernel(x, edge_index, W_l, W_r, att, W_res, bias):
    x = x.astype(jnp.float32)
    src = edge_index[0].astype(jnp.int32)
    dst = edge_index[1].astype(jnp.int32)
    loop = jnp.arange(N, dtype=jnp.int32)
    pad = jnp.zeros((E_ARR - E_SELF,), jnp.int32)
    srcp = jnp.concatenate([src, loop, pad])
    dstp = jnp.concatenate([dst, loop, pad])
    xl, xr, res = _mm_call(x, W_l.T, W_r.T, W_res.T, bias.reshape(1, D))
    accs, dens = _edge_call(xl, xr, att.astype(jnp.float32), srcp, dstp)
    return _fin_call(accs, dens, res)
